# baseline (speedup 1.0000x reference)
import functools

import jax
import jax.numpy as jnp
from jax import lax
from jax.experimental import pallas as pl
from jax.experimental.pallas import tpu as pltpu
from jax.experimental.pallas import tpu_sc as plsc

BATCH = 16384
VOC = 1000000
DIM = 64
H1 = 64
H2 = 32
NC = 2
NS = 16
NW = NC * NS
BPW = BATCH // NW
BLKP = 4096


def _tc_transform(uemb_t, iemb_t, W1):
    def body(ut_ref, it_ref, w1_ref, out_ref):
        au = lax.dot_general(ut_ref[...], w1_ref[0:DIM, :],
                             (((0,), (0,)), ((), ())),
                             preferred_element_type=jnp.float32)
        ai = lax.dot_general(it_ref[...], w1_ref[DIM:2 * DIM, :],
                             (((0,), (0,)), ((), ())),
                             preferred_element_type=jnp.float32)
        out_ref[...] = jnp.concatenate([au, ai], axis=1)

    grid = (pl.cdiv(VOC, BLKP),)
    return pl.pallas_call(
        body,
        grid=grid,
        in_specs=[
            pl.BlockSpec((DIM, BLKP), lambda i: (0, i)),
            pl.BlockSpec((DIM, BLKP), lambda i: (0, i)),
            pl.BlockSpec((2 * DIM, H1), lambda i: (0, 0)),
        ],
        out_specs=pl.BlockSpec((BLKP, 2 * H1), lambda i: (i, 0)),
        out_shape=jax.ShapeDtypeStruct((VOC, 2 * H1), jnp.float32),
    )(uemb_t, iemb_t, W1)


def _sc_gather_rows(user, item, t):
    mesh = plsc.VectorSubcoreMesh(core_axis_name="c", subcore_axis_name="s")

    @functools.partial(
        pl.kernel,
        mesh=mesh,
        out_type=(
            jax.ShapeDtypeStruct((BATCH, 2 * H1), jnp.float32),
            jax.ShapeDtypeStruct((BATCH, 2 * H1), jnp.float32),
        ),
        scratch_types=[
            pltpu.VMEM((BPW,), jnp.int32),
            pltpu.VMEM((BPW,), jnp.int32),
            pltpu.SemaphoreType.DMA,
            pltpu.SemaphoreType.DMA,
        ],
    )
    def k(user_hbm, item_hbm, t_hbm, uout_hbm, iout_hbm,
          uidx_v, iidx_v, usem, isem):
        wid = lax.axis_index("s") * NC + lax.axis_index("c")
        base = wid * BPW
        pltpu.sync_copy(user_hbm.at[pl.ds(base, BPW)], uidx_v)
        pltpu.sync_copy(item_hbm.at[pl.ds(base, BPW)], iidx_v)

        def issue(g, _):
            uv = uidx_v[pl.ds(g * 16, 16)]
            iv = iidx_v[pl.ds(g * 16, 16)]
            for l in range(16):
                j = base + g * 16 + l
                pltpu.async_copy(t_hbm.at[uv[l]], uout_hbm.at[j], usem)
                pltpu.async_copy(t_hbm.at[iv[l]], iout_hbm.at[j], isem)
            return _

        lax.fori_loop(0, BPW // 16, issue, 0)
        pltpu.make_async_copy(
            t_hbm.at[pl.ds(0, BPW)], uout_hbm.at[pl.ds(base, BPW)],
            usem).wait()
        pltpu.make_async_copy(
            t_hbm.at[pl.ds(0, BPW)], iout_hbm.at[pl.ds(base, BPW)],
            isem).wait()

    return k(user, item, t)


def _tc_finish(uf, itf, W2, b1, b2):
    BLK = 4096

    def body(u_ref, i_ref, w2_ref, b1_ref, b2_ref, out_ref):
        a = u_ref[:, 0:H1] + i_ref[:, H1:2 * H1] + b1_ref[...]
        out = jnp.dot(a, w2_ref[...],
                      preferred_element_type=jnp.float32) + b2_ref[...]
        out_ref[...] = jnp.maximum(out, 0.0)

    return pl.pallas_call(
        body,
        grid=(BATCH // BLK,),
        in_specs=[
            pl.BlockSpec((BLK, 2 * H1), lambda i: (i, 0)),
            pl.BlockSpec((BLK, 2 * H1), lambda i: (i, 0)),
            pl.BlockSpec((H1, H2), lambda i: (0, 0)),
            pl.BlockSpec((1, H1), lambda i: (0, 0)),
            pl.BlockSpec((1, H2), lambda i: (0, 0)),
        ],
        out_specs=pl.BlockSpec((BLK, H2), lambda i: (i, 0)),
        out_shape=jax.ShapeDtypeStruct((BATCH, H2), jnp.float32),
    )(uf, itf, W2, b1.reshape(1, H1), b2.reshape(1, H2))


def kernel(user, item, user_emb, item_emb, W1, b1, W2, b2):
    t = _tc_transform(user_emb.T, item_emb.T, W1)
    uf, itf = _sc_gather_rows(user, item, t)
    return _tc_finish(uf, itf, W2, b1, b2)


# speedup vs baseline: 1.8799x; 1.8799x over previous
import functools

import jax
import jax.numpy as jnp
from jax import lax
from jax.experimental import pallas as pl
from jax.experimental.pallas import tpu as pltpu
from jax.experimental.pallas import tpu_sc as plsc

BATCH = 16384
VOC = 1000000
DIM = 64
H1 = 64
H2 = 32
NC = 2
NS = 16
NW = NC * NS
BPW = BATCH // NW
BLKP = 4096


def _tc_transform(uemb_t, iemb_t, W1):
    def body(ut_ref, it_ref, w1_ref, out_ref):
        au = lax.dot_general(ut_ref[...], w1_ref[0:DIM, :],
                             (((0,), (0,)), ((), ())),
                             preferred_element_type=jnp.float32)
        ai = lax.dot_general(it_ref[...], w1_ref[DIM:2 * DIM, :],
                             (((0,), (0,)), ((), ())),
                             preferred_element_type=jnp.float32)
        out_ref[...] = jnp.concatenate([au, ai], axis=1)

    grid = (pl.cdiv(VOC, BLKP),)
    return pl.pallas_call(
        body,
        grid=grid,
        in_specs=[
            pl.BlockSpec((DIM, BLKP), lambda i: (0, i)),
            pl.BlockSpec((DIM, BLKP), lambda i: (0, i)),
            pl.BlockSpec((2 * DIM, H1), lambda i: (0, 0)),
        ],
        out_specs=pl.BlockSpec((BLKP, 2 * H1), lambda i: (i, 0)),
        out_shape=jax.ShapeDtypeStruct((VOC, 2 * H1), jnp.float32),
    )(uemb_t, iemb_t, W1)


CH = 128
NCH = BPW // CH


def _sc_gather_rows(user, item, t):
    mesh = plsc.VectorSubcoreMesh(core_axis_name="c", subcore_axis_name="s")

    @functools.partial(
        pl.kernel,
        mesh=mesh,
        out_type=(
            jax.ShapeDtypeStruct((BATCH, 2 * H1), jnp.float32),
            jax.ShapeDtypeStruct((BATCH, 2 * H1), jnp.float32),
        ),
        scratch_types=[
            pltpu.VMEM((BPW,), jnp.int32),
            pltpu.VMEM((BPW,), jnp.int32),
            pltpu.VMEM((2, CH, 2 * H1), jnp.float32),
            pltpu.VMEM((2, CH, 2 * H1), jnp.float32),
            pltpu.SemaphoreType.DMA,
            pltpu.SemaphoreType.DMA,
            pltpu.SemaphoreType.DMA,
        ],
    )
    def k(user_hbm, item_hbm, t_hbm, uout_hbm, iout_hbm,
          uidx_v, iidx_v, urows_v, irows_v, gsem, isem, wsem):
        wid = lax.axis_index("s") * NC + lax.axis_index("c")
        base = wid * BPW
        pltpu.sync_copy(user_hbm.at[pl.ds(base, BPW)], uidx_v)
        pltpu.sync_copy(item_hbm.at[pl.ds(base, BPW)], iidx_v)

        wbs = []
        for c in range(NCH):
            buf = c % 2
            if c >= 2:
                wbs.pop(0).wait()
                wbs.pop(0).wait()

            def issue(g, _, c=c, buf=buf):
                uv = uidx_v[pl.ds(c * CH + g * 16, 16)]
                iv = iidx_v[pl.ds(c * CH + g * 16, 16)]
                for l in range(16):
                    j = g * 16 + l
                    pltpu.async_copy(
                        t_hbm.at[uv[l]], urows_v.at[buf, j], gsem)
                    pltpu.async_copy(
                        t_hbm.at[iv[l]], irows_v.at[buf, j], isem)
                return _

            lax.fori_loop(0, CH // 16, issue, 0)
            pltpu.make_async_copy(
                t_hbm.at[pl.ds(0, CH)], urows_v.at[buf], gsem).wait()
            pltpu.make_async_copy(
                t_hbm.at[pl.ds(0, CH)], irows_v.at[buf], isem).wait()
            dst = pl.ds(base + c * CH, CH)
            wbs.append(pltpu.async_copy(urows_v.at[buf], uout_hbm.at[dst], wsem))
            wbs.append(pltpu.async_copy(irows_v.at[buf], iout_hbm.at[dst], wsem))
        for wb in wbs:
            wb.wait()

    return k(user, item, t)


def _tc_finish(uf, itf, W2, b1, b2):
    BLK = 4096

    def body(u_ref, i_ref, w2_ref, b1_ref, b2_ref, out_ref):
        a = u_ref[:, 0:H1] + i_ref[:, H1:2 * H1] + b1_ref[...]
        out = jnp.dot(a, w2_ref[...],
                      preferred_element_type=jnp.float32) + b2_ref[...]
        out_ref[...] = jnp.maximum(out, 0.0)

    return pl.pallas_call(
        body,
        grid=(BATCH // BLK,),
        in_specs=[
            pl.BlockSpec((BLK, 2 * H1), lambda i: (i, 0)),
            pl.BlockSpec((BLK, 2 * H1), lambda i: (i, 0)),
            pl.BlockSpec((H1, H2), lambda i: (0, 0)),
            pl.BlockSpec((1, H1), lambda i: (0, 0)),
            pl.BlockSpec((1, H2), lambda i: (0, 0)),
        ],
        out_specs=pl.BlockSpec((BLK, H2), lambda i: (i, 0)),
        out_shape=jax.ShapeDtypeStruct((BATCH, H2), jnp.float32),
    )(uf, itf, W2, b1.reshape(1, H1), b2.reshape(1, H2))


def kernel(user, item, user_emb, item_emb, W1, b1, W2, b2):
    t = _tc_transform(user_emb.T, item_emb.T, W1)
    uf, itf = _sc_gather_rows(user, item, t)
    return _tc_finish(uf, itf, W2, b1, b2)


# speedup vs baseline: 2.1447x; 1.1408x over previous
import functools

import jax
import jax.numpy as jnp
from jax import lax
from jax.experimental import pallas as pl
from jax.experimental.pallas import tpu as pltpu
from jax.experimental.pallas import tpu_sc as plsc

BATCH = 16384
VOC = 1000000
DIM = 64
H1 = 64
H2 = 32
NC = 2
NS = 16
NW = NC * NS
BPW = BATCH // NW
BLKP = 8192


def _tc_transform(uemb_t, iemb_t, W1):
    def body(ut_ref, it_ref, w1_ref, out_ref):
        au = lax.dot_general(ut_ref[...], w1_ref[0:DIM, :],
                             (((0,), (0,)), ((), ())),
                             preferred_element_type=jnp.float32)
        ai = lax.dot_general(it_ref[...], w1_ref[DIM:2 * DIM, :],
                             (((0,), (0,)), ((), ())),
                             preferred_element_type=jnp.float32)
        out_ref[...] = jnp.concatenate([au, ai], axis=1)

    grid = (pl.cdiv(VOC, BLKP),)
    return pl.pallas_call(
        body,
        grid=grid,
        in_specs=[
            pl.BlockSpec((DIM, BLKP), lambda i: (0, i)),
            pl.BlockSpec((DIM, BLKP), lambda i: (0, i)),
            pl.BlockSpec((2 * DIM, H1), lambda i: (0, 0)),
        ],
        out_specs=pl.BlockSpec((BLKP, 2 * H1), lambda i: (i, 0)),
        out_shape=jax.ShapeDtypeStruct((VOC, 2 * H1), jnp.float32),
    )(uemb_t, iemb_t, W1)


CH = 128
NCH = BPW // CH


def _sc_gather_rows(user, item, t):
    mesh = plsc.VectorSubcoreMesh(core_axis_name="c", subcore_axis_name="s")

    @functools.partial(
        pl.kernel,
        mesh=mesh,
        out_type=(
            jax.ShapeDtypeStruct((BATCH, 2 * H1), jnp.float32),
            jax.ShapeDtypeStruct((BATCH, 2 * H1), jnp.float32),
        ),
        scratch_types=[
            pltpu.VMEM((BPW,), jnp.int32),
            pltpu.VMEM((BPW,), jnp.int32),
            pltpu.VMEM((2, CH, 2 * H1), jnp.float32),
            pltpu.VMEM((2, CH, 2 * H1), jnp.float32),
            pltpu.SemaphoreType.DMA,
            pltpu.SemaphoreType.DMA,
            pltpu.SemaphoreType.DMA,
        ],
    )
    def k(user_hbm, item_hbm, t_hbm, uout_hbm, iout_hbm,
          uidx_v, iidx_v, urows_v, irows_v, gsem, isem, wsem):
        wid = lax.axis_index("s") * NC + lax.axis_index("c")
        base = wid * BPW
        pltpu.sync_copy(user_hbm.at[pl.ds(base, BPW)], uidx_v)
        pltpu.sync_copy(item_hbm.at[pl.ds(base, BPW)], iidx_v)

        wbs = []
        for c in range(NCH):
            buf = c % 2
            if c >= 2:
                wbs.pop(0).wait()
                wbs.pop(0).wait()

            def issue(g, _, c=c, buf=buf):
                uv = uidx_v[pl.ds(c * CH + g * 16, 16)]
                iv = iidx_v[pl.ds(c * CH + g * 16, 16)]
                for l in range(16):
                    j = g * 16 + l
                    pltpu.async_copy(
                        t_hbm.at[uv[l]], urows_v.at[buf, j], gsem)
                    pltpu.async_copy(
                        t_hbm.at[iv[l]], irows_v.at[buf, j], isem)
                return _

            lax.fori_loop(0, CH // 16, issue, 0)
            pltpu.make_async_copy(
                t_hbm.at[pl.ds(0, CH)], urows_v.at[buf], gsem).wait()
            pltpu.make_async_copy(
                t_hbm.at[pl.ds(0, CH)], irows_v.at[buf], isem).wait()
            dst = pl.ds(base + c * CH, CH)
            wbs.append(pltpu.async_copy(urows_v.at[buf], uout_hbm.at[dst], wsem))
            wbs.append(pltpu.async_copy(irows_v.at[buf], iout_hbm.at[dst], wsem))
        for wb in wbs:
            wb.wait()

    return k(user, item, t)


def _tc_finish(uf, itf, W2, b1, b2):
    BLK = 4096

    def body(u_ref, i_ref, w2_ref, b1_ref, b2_ref, out_ref):
        a = u_ref[:, 0:H1] + i_ref[:, H1:2 * H1] + b1_ref[...]
        out = jnp.dot(a, w2_ref[...],
                      preferred_element_type=jnp.float32) + b2_ref[...]
        out_ref[...] = jnp.maximum(out, 0.0)

    return pl.pallas_call(
        body,
        grid=(BATCH // BLK,),
        in_specs=[
            pl.BlockSpec((BLK, 2 * H1), lambda i: (i, 0)),
            pl.BlockSpec((BLK, 2 * H1), lambda i: (i, 0)),
            pl.BlockSpec((H1, H2), lambda i: (0, 0)),
            pl.BlockSpec((1, H1), lambda i: (0, 0)),
            pl.BlockSpec((1, H2), lambda i: (0, 0)),
        ],
        out_specs=pl.BlockSpec((BLK, H2), lambda i: (i, 0)),
        out_shape=jax.ShapeDtypeStruct((BATCH, H2), jnp.float32),
    )(uf, itf, W2, b1.reshape(1, H1), b2.reshape(1, H2))


def kernel(user, item, user_emb, item_emb, W1, b1, W2, b2):
    t = _tc_transform(user_emb.T, item_emb.T, W1)
    uf, itf = _sc_gather_rows(user, item, t)
    return _tc_finish(uf, itf, W2, b1, b2)


# speedup vs baseline: 2.2847x; 1.0653x over previous
import functools

import jax
import jax.numpy as jnp
from jax import lax
from jax.experimental import pallas as pl
from jax.experimental.pallas import tpu as pltpu
from jax.experimental.pallas import tpu_sc as plsc

BATCH = 16384
VOC = 1000000
DIM = 64
H1 = 64
H2 = 32
NC = 2
NS = 16
NW = NC * NS
BPW = BATCH // NW
BLKP = 16384


def _tc_transform(uemb_t, iemb_t, W1):
    def body(ut_ref, it_ref, w1_ref, out_ref):
        au = lax.dot_general(ut_ref[...], w1_ref[0:DIM, :],
                             (((0,), (0,)), ((), ())),
                             preferred_element_type=jnp.float32)
        ai = lax.dot_general(it_ref[...], w1_ref[DIM:2 * DIM, :],
                             (((0,), (0,)), ((), ())),
                             preferred_element_type=jnp.float32)
        out_ref[...] = jnp.concatenate([au, ai], axis=1)

    grid = (pl.cdiv(VOC, BLKP),)
    return pl.pallas_call(
        body,
        grid=grid,
        in_specs=[
            pl.BlockSpec((DIM, BLKP), lambda i: (0, i)),
            pl.BlockSpec((DIM, BLKP), lambda i: (0, i)),
            pl.BlockSpec((2 * DIM, H1), lambda i: (0, 0)),
        ],
        out_specs=pl.BlockSpec((BLKP, 2 * H1), lambda i: (i, 0)),
        out_shape=jax.ShapeDtypeStruct((VOC, 2 * H1), jnp.float32),
    )(uemb_t, iemb_t, W1)


CH = 128
NCH = BPW // CH


def _sc_gather_rows(user, item, t):
    mesh = plsc.VectorSubcoreMesh(core_axis_name="c", subcore_axis_name="s")

    @functools.partial(
        pl.kernel,
        mesh=mesh,
        out_type=(
            jax.ShapeDtypeStruct((BATCH, 2 * H1), jnp.float32),
            jax.ShapeDtypeStruct((BATCH, 2 * H1), jnp.float32),
        ),
        scratch_types=[
            pltpu.VMEM((BPW,), jnp.int32),
            pltpu.VMEM((BPW,), jnp.int32),
            pltpu.VMEM((2, CH, 2 * H1), jnp.float32),
            pltpu.VMEM((2, CH, 2 * H1), jnp.float32),
            pltpu.SemaphoreType.DMA,
            pltpu.SemaphoreType.DMA,
            pltpu.SemaphoreType.DMA,
        ],
    )
    def k(user_hbm, item_hbm, t_hbm, uout_hbm, iout_hbm,
          uidx_v, iidx_v, urows_v, irows_v, gsem, isem, wsem):
        wid = lax.axis_index("s") * NC + lax.axis_index("c")
        base = wid * BPW
        pltpu.sync_copy(user_hbm.at[pl.ds(base, BPW)], uidx_v)
        pltpu.sync_copy(item_hbm.at[pl.ds(base, BPW)], iidx_v)

        wbs = []
        for c in range(NCH):
            buf = c % 2
            if c >= 2:
                wbs.pop(0).wait()
                wbs.pop(0).wait()

            def issue(g, _, c=c, buf=buf):
                uv = uidx_v[pl.ds(c * CH + g * 16, 16)]
                iv = iidx_v[pl.ds(c * CH + g * 16, 16)]
                for l in range(16):
                    j = g * 16 + l
                    pltpu.async_copy(
                        t_hbm.at[uv[l]], urows_v.at[buf, j], gsem)
                    pltpu.async_copy(
                        t_hbm.at[iv[l]], irows_v.at[buf, j], isem)
                return _

            lax.fori_loop(0, CH // 16, issue, 0)
            pltpu.make_async_copy(
                t_hbm.at[pl.ds(0, CH)], urows_v.at[buf], gsem).wait()
            pltpu.make_async_copy(
                t_hbm.at[pl.ds(0, CH)], irows_v.at[buf], isem).wait()
            dst = pl.ds(base + c * CH, CH)
            wbs.append(pltpu.async_copy(urows_v.at[buf], uout_hbm.at[dst], wsem))
            wbs.append(pltpu.async_copy(irows_v.at[buf], iout_hbm.at[dst], wsem))
        for wb in wbs:
            wb.wait()

    return k(user, item, t)


def _tc_finish(uf, itf, W2, b1, b2):
    BLK = 4096

    def body(u_ref, i_ref, w2_ref, b1_ref, b2_ref, out_ref):
        a = u_ref[:, 0:H1] + i_ref[:, H1:2 * H1] + b1_ref[...]
        out = jnp.dot(a, w2_ref[...],
                      preferred_element_type=jnp.float32) + b2_ref[...]
        out_ref[...] = jnp.maximum(out, 0.0)

    return pl.pallas_call(
        body,
        grid=(BATCH // BLK,),
        in_specs=[
            pl.BlockSpec((BLK, 2 * H1), lambda i: (i, 0)),
            pl.BlockSpec((BLK, 2 * H1), lambda i: (i, 0)),
            pl.BlockSpec((H1, H2), lambda i: (0, 0)),
            pl.BlockSpec((1, H1), lambda i: (0, 0)),
            pl.BlockSpec((1, H2), lambda i: (0, 0)),
        ],
        out_specs=pl.BlockSpec((BLK, H2), lambda i: (i, 0)),
        out_shape=jax.ShapeDtypeStruct((BATCH, H2), jnp.float32),
    )(uf, itf, W2, b1.reshape(1, H1), b2.reshape(1, H2))


def kernel(user, item, user_emb, item_emb, W1, b1, W2, b2):
    t = _tc_transform(user_emb.T, item_emb.T, W1)
    uf, itf = _sc_gather_rows(user, item, t)
    return _tc_finish(uf, itf, W2, b1, b2)
